# baseline (speedup 1.0000x reference)
import functools

import jax
import jax.numpy as jnp
from jax import lax
from jax.experimental import pallas as pl
from jax.experimental.pallas import tpu as pltpu
from jax.experimental.pallas import tpu_sc as plsc

_NC = 2
_NS = 16
_NW = _NC * _NS
_L = 16

_CELLS = 16 * 16
_CPW = _CELLS // _NW
_NTC = 32
_HTC = _NTC // 2
_PAIR = 2 * 4096
_SLAB = _NTC * 8 * 128
_HSLAB = _SLAB // 2
_TREP = 2048 * _L


def _worker_body(img_hbm, ent_hbm, col_hbm, out_hbm,
                 ent_v, col_v, pairs_v, out_v0, out_v1, sem0, sem1):
    wid = lax.axis_index("s") * _NC + lax.axis_index("c")
    pltpu.sync_copy(ent_hbm, ent_v)
    pltpu.sync_copy(col_hbm, col_v)
    lanes = lax.iota(jnp.int32, _L)
    out_bufs = (out_v0, out_v1)
    sems = (sem0, sem1)

    def ij_body(l, carry):
        ij = wid * _CPW + l
        pltpu.sync_copy(img_hbm.at[pl.ds(ij * _PAIR, _PAIR)], pairs_v)
        pending = [None, None]
        for tr in range(8):
            tab_v = ent_v if tr < 4 else col_v
            coff = (tr * 8 if tr < 4 else (tr - 4) * 8) * _L
            poff = 0 if tr < 4 else 128
            for h in range(2):
                b = (tr * 2 + h) % 2
                if pending[b] is not None:
                    pending[b].wait()
                    pending[b] = None
                out_v = out_bufs[b]

                def tc_body(tc0, carry2, tab_v=tab_v, coff=coff, poff=poff,
                            out_v=out_v, h=h):
                    tc = h * _HTC + tc0 * 2
                    prev = None
                    prev_sidx = 0
                    for g in range(16):
                        po = tc * 256 + (g // 8) * 256 + poff + (g % 8) * 16
                        idx16 = pairs_v[pl.ds(po, _L)]
                        base = idx16 * (32 * _L) + (lanes + coff)
                        sidx = tc0 * 2048 + (g // 8) * 1024 + (g % 8) * 16
                        cur = []
                        for dr in range(8):
                            cur.append(
                                plsc.load_gather(tab_v, [base + dr * _L])
                            )
                            if prev is not None:
                                out_v[pl.ds(prev_sidx + dr * 128, _L)] = (
                                    prev[dr]
                                )
                        prev, prev_sidx = cur, sidx
                    for dr in range(8):
                        out_v[pl.ds(prev_sidx + dr * 128, _L)] = prev[dr]
                    return carry2

                lax.fori_loop(0, _HTC // 2, tc_body, 0)
                pending[b] = pltpu.async_copy(
                    out_v,
                    out_hbm.at[
                        pl.ds(ij * (8 * _SLAB) + tr * _SLAB + h * _HSLAB,
                              _HSLAB)
                    ],
                    sems[b],
                )
        for b in range(2):
            if pending[b] is not None:
                pending[b].wait()
        return carry

    lax.fori_loop(0, _CPW, ij_body, 0)


_sc_mesh = plsc.VectorSubcoreMesh(core_axis_name="c", subcore_axis_name="s")

_sc_lookup = functools.partial(
    pl.kernel,
    mesh=_sc_mesh,
    out_type=jax.ShapeDtypeStruct((_CELLS * 8 * _SLAB,), jnp.float32),
    scratch_types=[
        pltpu.VMEM((_TREP,), jnp.float32),
        pltpu.VMEM((_TREP,), jnp.float32),
        pltpu.VMEM((_PAIR,), jnp.int32),
        pltpu.VMEM((_HSLAB,), jnp.float32),
        pltpu.VMEM((_HSLAB,), jnp.float32),
        pltpu.SemaphoreType.DMA,
        pltpu.SemaphoreType.DMA,
    ],
    compiler_params=pltpu.CompilerParams(
        needs_layout_passes=False, use_tc_tiling_on_sc=False
    ),
)(_worker_body)


def kernel(img, entity_table, color_table):
    img_p = (
        img.transpose(1, 2, 3, 0)
        .reshape(16, 16, 2, 32, 128)
        .transpose(0, 1, 3, 2, 4)
        .reshape(-1)
    )
    ent = jnp.repeat(entity_table.reshape(-1), _L)
    col = jnp.repeat(color_table.reshape(-1), _L)
    outp = _sc_lookup(img_p, ent, col)
    out6 = outp.reshape(16, 16, 8, 32, 8, 128)
    return out6.transpose(3, 5, 0, 1, 2, 4).reshape(4096, 16, 16, 64)


# speedup vs baseline: 1.0673x; 1.0673x over previous
import functools

import jax
import jax.numpy as jnp
from jax import lax
from jax.experimental import pallas as pl
from jax.experimental.pallas import tpu as pltpu
from jax.experimental.pallas import tpu_sc as plsc

_NC = 2
_NS = 16
_NW = _NC * _NS
_L = 16

_CELLS = 16 * 16
_CPW = _CELLS // _NW
_NTC = 32
_HTC = _NTC // 2
_PAIR = 2 * 4096
_SLAB = _NTC * 8 * 128
_HSLAB = _SLAB // 2
_TREP = 2048 * _L


def _worker_body(img_hbm, ent_hbm, col_hbm, out_hbm,
                 ent_v, col_v, pairs_v, out_v0, out_v1, sem0, sem1):
    wid = lax.axis_index("s") * _NC + lax.axis_index("c")
    pltpu.sync_copy(ent_hbm, ent_v)
    pltpu.sync_copy(col_hbm, col_v)
    lanes = lax.iota(jnp.int32, _L)
    out_bufs = (out_v0, out_v1)
    sems = (sem0, sem1)

    def ij_body(l, carry):
        ij = wid * _CPW + l
        pltpu.sync_copy(img_hbm.at[pl.ds(ij * _PAIR, _PAIR)], pairs_v)
        pending = [None, None]
        for tr in range(8):
            tab_v = ent_v if tr < 4 else col_v
            coff = (tr * 8 if tr < 4 else (tr - 4) * 8) * _L
            poff = 0 if tr < 4 else 128
            for h in range(2):
                b = (tr * 2 + h) % 2
                if pending[b] is not None:
                    pending[b].wait()
                    pending[b] = None
                out_v = out_bufs[b]

                def tc_body(tc0, carry2, tab_v=tab_v, coff=coff, poff=poff,
                            out_v=out_v, h=h):
                    tc = h * _HTC + tc0
                    prev = None
                    prev_sidx = 0
                    for g in range(8):
                        idx16 = pairs_v[pl.ds(tc * 256 + poff + g * 16, _L)]
                        base = idx16 * (32 * _L) + (lanes + coff)
                        sidx = tc0 * 1024 + g * 16
                        cur = []
                        for dr in range(8):
                            cur.append(
                                plsc.load_gather(tab_v, [base + dr * _L])
                            )
                            if prev is not None:
                                out_v[pl.ds(prev_sidx + dr * 128, _L)] = (
                                    prev[dr]
                                )
                        prev, prev_sidx = cur, sidx
                    for dr in range(8):
                        out_v[pl.ds(prev_sidx + dr * 128, _L)] = prev[dr]
                    return carry2

                lax.fori_loop(0, _HTC, tc_body, 0)
                pending[b] = pltpu.async_copy(
                    out_v,
                    out_hbm.at[
                        pl.ds(ij * (8 * _SLAB) + tr * _SLAB + h * _HSLAB,
                              _HSLAB)
                    ],
                    sems[b],
                )
        for b in range(2):
            if pending[b] is not None:
                pending[b].wait()
        return carry

    lax.fori_loop(0, _CPW, ij_body, 0)


_sc_mesh = plsc.VectorSubcoreMesh(core_axis_name="c", subcore_axis_name="s")

_sc_lookup = functools.partial(
    pl.kernel,
    mesh=_sc_mesh,
    out_type=jax.ShapeDtypeStruct((_CELLS * 8 * _SLAB,), jnp.float32),
    scratch_types=[
        pltpu.VMEM((_TREP,), jnp.float32),
        pltpu.VMEM((_TREP,), jnp.float32),
        pltpu.VMEM((_PAIR,), jnp.int32),
        pltpu.VMEM((_HSLAB,), jnp.float32),
        pltpu.VMEM((_HSLAB,), jnp.float32),
        pltpu.SemaphoreType.DMA,
        pltpu.SemaphoreType.DMA,
    ],
    compiler_params=pltpu.CompilerParams(
        needs_layout_passes=False, use_tc_tiling_on_sc=False
    ),
)(_worker_body)


def kernel(img, entity_table, color_table):
    img_p = (
        img.transpose(1, 2, 3, 0)
        .reshape(16, 16, 2, 32, 128)
        .transpose(0, 1, 3, 2, 4)
        .reshape(-1)
    )
    ent = jnp.repeat(entity_table.reshape(-1), _L)
    col = jnp.repeat(color_table.reshape(-1), _L)
    outp = _sc_lookup(img_p, ent, col)
    out6 = outp.reshape(16, 16, 8, 32, 8, 128)
    return out6.transpose(3, 5, 0, 1, 2, 4).reshape(4096, 16, 16, 64)


# speedup vs baseline: 1.3578x; 1.2722x over previous
import functools

import jax
import jax.numpy as jnp
from jax import lax
from jax.experimental import pallas as pl
from jax.experimental.pallas import tpu as pltpu
from jax.experimental.pallas import tpu_sc as plsc

_NC = 2
_NS = 16
_NW = _NC * _NS
_L = 16

_CELLS = 16 * 16
_CPW = _CELLS // _NW
_NTC = 32
_HTC = _NTC // 2
_PAIR = 2 * 4096
_SLAB = _NTC * 8 * 128
_HSLAB = _SLAB // 2
_TREP = 2048 * _L


def _worker_body(img_hbm, ent_hbm, col_hbm, out_hbm,
                 ent_v, col_v, pairs_v, out_v0, out_v1, sem0, sem1):
    wid = lax.axis_index("s") * _NC + lax.axis_index("c")
    pltpu.sync_copy(ent_hbm, ent_v)
    pltpu.sync_copy(col_hbm, col_v)
    lanes = lax.iota(jnp.int32, _L)
    out_bufs = (out_v0, out_v1)
    sems = (sem0, sem1)

    def ij_body(l, carry):
        ij = wid * _CPW + l
        pltpu.sync_copy(img_hbm.at[pl.ds(ij * _PAIR, _PAIR)], pairs_v)
        pending = [None, None]
        for tr in range(8):
            tab_v = ent_v if tr < 4 else col_v
            coff = (tr * 8 if tr < 4 else (tr - 4) * 8) * _L
            poff = 0 if tr < 4 else 128
            for h in range(2):
                b = (tr * 2 + h) % 2
                if pending[b] is not None:
                    pending[b].wait()
                    pending[b] = None
                out_v = out_bufs[b]

                def tc_body(tc0, tab_v=tab_v, coff=coff, poff=poff,
                            out_v=out_v, h=h):
                    tc = h * _HTC + tc0
                    prev = None
                    prev_sidx = 0
                    for g in range(8):
                        idx16 = pairs_v[pl.ds(tc * 256 + poff + g * 16, _L)]
                        base = idx16 * (32 * _L) + (lanes + coff)
                        sidx = tc0 * 1024 + g * 16
                        cur = []
                        for dr in range(8):
                            cur.append(
                                plsc.load_gather(tab_v, [base + dr * _L])
                            )
                            if prev is not None:
                                out_v[pl.ds(prev_sidx + dr * 128, _L)] = (
                                    prev[dr]
                                )
                        prev, prev_sidx = cur, sidx
                    for dr in range(8):
                        out_v[pl.ds(prev_sidx + dr * 128, _L)] = prev[dr]

                plsc.parallel_loop(0, _HTC, 1)(tc_body)
                pending[b] = pltpu.async_copy(
                    out_v,
                    out_hbm.at[
                        pl.ds(ij * (8 * _SLAB) + tr * _SLAB + h * _HSLAB,
                              _HSLAB)
                    ],
                    sems[b],
                )
        for b in range(2):
            if pending[b] is not None:
                pending[b].wait()
        return carry

    lax.fori_loop(0, _CPW, ij_body, 0)


_sc_mesh = plsc.VectorSubcoreMesh(core_axis_name="c", subcore_axis_name="s")

_sc_lookup = functools.partial(
    pl.kernel,
    mesh=_sc_mesh,
    out_type=jax.ShapeDtypeStruct((_CELLS * 8 * _SLAB,), jnp.float32),
    scratch_types=[
        pltpu.VMEM((_TREP,), jnp.float32),
        pltpu.VMEM((_TREP,), jnp.float32),
        pltpu.VMEM((_PAIR,), jnp.int32),
        pltpu.VMEM((_HSLAB,), jnp.float32),
        pltpu.VMEM((_HSLAB,), jnp.float32),
        pltpu.SemaphoreType.DMA,
        pltpu.SemaphoreType.DMA,
    ],
    compiler_params=pltpu.CompilerParams(
        needs_layout_passes=False, use_tc_tiling_on_sc=False
    ),
)(_worker_body)


def kernel(img, entity_table, color_table):
    img_p = (
        img.transpose(1, 2, 3, 0)
        .reshape(16, 16, 2, 32, 128)
        .transpose(0, 1, 3, 2, 4)
        .reshape(-1)
    )
    ent = jnp.repeat(entity_table.reshape(-1), _L)
    col = jnp.repeat(color_table.reshape(-1), _L)
    outp = _sc_lookup(img_p, ent, col)
    out6 = outp.reshape(16, 16, 8, 32, 8, 128)
    return out6.transpose(3, 5, 0, 1, 2, 4).reshape(4096, 16, 16, 64)
